# Initial kernel scaffold; baseline (speedup 1.0000x reference)
import jax
import jax.numpy as jnp
from jax.experimental import pallas as pl


def kernel(r, spline_values, spline_derivatives, spline_spacing, cutoff):
    raise NotImplementedError("write your pallas kernel here")


# speedup vs baseline: 3.1233x; 3.1233x over previous
import functools

import jax
import jax.numpy as jnp
from jax import lax
from jax.experimental import pallas as pl
from jax.experimental.pallas import tpu as pltpu
from jax.experimental.pallas import tpu_sc as plsc

_NR = 320000
_NROWS = 1026
_NC = 128
_NCG = 4
_NEG = 8
_CW = _NC // _NCG
_TCOLS = 2 * _CW
_TLEN = _NROWS * _TCOLS
_EPW = _NR // _NEG
_CHUNK = 400
_NCHUNK = _EPW // _CHUNK


def _body(params_h, table_h, r_h, out_h, table_v, params_v, r_v, n_v, h_v, out_v):
    cid = lax.axis_index("c")
    sid = lax.axis_index("s")
    wid = sid * 2 + cid
    g = wid % _NCG
    eg = wid // _NCG

    pltpu.sync_copy(table_h.at[g], table_v)
    pltpu.sync_copy(params_h, params_v)
    d_vec = params_v[0]
    cut_vec = params_v[1]

    def chunk_body(c, carry):
        row0 = eg * _EPW + c * _CHUNK
        pltpu.sync_copy(r_h.at[pl.ds(row0, _CHUNK)], r_v)

        def prep(i, carry2):
            rr = r_v[pl.ds(i * 16, 16)]
            x = jnp.minimum(jnp.maximum(rr, 0.0), cut_vec)
            q = x / d_vec
            ni = q.astype(jnp.int32)
            t = (x - ni.astype(jnp.float32) * d_vec) / d_vec
            t2 = t * t
            t3 = t2 * t
            n_v[pl.ds(i * 16, 16)] = ni * _TCOLS
            h_v[0, pl.ds(i * 16, 16)] = 2.0 * t3 - 3.0 * t2 + 1.0
            h_v[1, pl.ds(i * 16, 16)] = t3 - 2.0 * t2 + t
            h_v[2, pl.ds(i * 16, 16)] = 3.0 * t2 - 2.0 * t3
            h_v[3, pl.ds(i * 16, 16)] = t3 - t2
            return carry2

        lax.fori_loop(0, _CHUNK // 16, prep, 0)

        def elem_block(b, carry2):
            base = b * 16
            offv = n_v[pl.ds(base, 16)]
            h0v = h_v[0, pl.ds(base, 16)]
            h1v = h_v[1, pl.ds(base, 16)]
            h2v = h_v[2, pl.ds(base, 16)]
            h3v = h_v[3, pl.ds(base, 16)]
            for k in range(16):
                off = offv[k]
                a00 = h0v[k]
                a10 = h1v[k]
                a01 = h2v[k]
                a11 = h3v[k]
                for j in range(_CW // 16):
                    vn = table_v[pl.ds(off + 16 * j, 16)]
                    dn = table_v[pl.ds(off + _CW + 16 * j, 16)]
                    vn1 = table_v[pl.ds(off + _TCOLS + 16 * j, 16)]
                    dn1 = table_v[pl.ds(off + _TCOLS + _CW + 16 * j, 16)]
                    acc = a00 * vn + a10 * dn + a01 * vn1 + a11 * dn1
                    out_v[base + k, pl.ds(16 * j, 16)] = acc
            return carry2

        lax.fori_loop(0, _CHUNK // 16, elem_block, 0)
        pltpu.sync_copy(out_v, out_h.at[pl.ds(row0, _CHUNK), pl.ds(g * _CW, _CW)])
        return carry

    lax.fori_loop(0, _NCHUNK, chunk_body, 0)


def kernel(r, spline_values, spline_derivatives, spline_spacing, cutoff):
    assert r.shape == (_NR,) and spline_values.shape == (_NROWS, _NC)
    spacing = jnp.asarray(spline_spacing, jnp.float32)
    dscaled = spline_derivatives.astype(jnp.float32) * spacing
    groups = []
    for g in range(_NCG):
        sl = slice(g * _CW, (g + 1) * _CW)
        tg = jnp.concatenate(
            [spline_values[:, sl].astype(jnp.float32), dscaled[:, sl]], axis=1
        )
        groups.append(tg.reshape(-1))
    table = jnp.stack(groups)
    params = jnp.stack(
        [
            jnp.full((16,), spacing, jnp.float32),
            jnp.full((16,), jnp.asarray(cutoff, jnp.float32), jnp.float32),
        ]
    )

    mesh = plsc.VectorSubcoreMesh(core_axis_name="c", subcore_axis_name="s")
    run = pl.kernel(
        _body,
        out_type=jax.ShapeDtypeStruct((_NR, _NC), jnp.float32),
        mesh=mesh,
        compiler_params=pltpu.CompilerParams(use_tc_tiling_on_sc=False),
        scratch_types=[
            pltpu.VMEM((_TLEN,), jnp.float32),
            pltpu.VMEM((2, 16), jnp.float32),
            pltpu.VMEM((_CHUNK,), jnp.float32),
            pltpu.VMEM((_CHUNK,), jnp.int32),
            pltpu.VMEM((4, _CHUNK), jnp.float32),
            pltpu.VMEM((_CHUNK, _CW), jnp.float32),
        ],
    )
    return run(params, table, r.astype(jnp.float32))
